# Initial kernel scaffold; baseline (speedup 1.0000x reference)
import jax
import jax.numpy as jnp
from jax.experimental import pallas as pl


def kernel(x, adj, length, W1, b1, W2, b2, Wlin, blin):
    raise NotImplementedError("write your pallas kernel here")


# speedup vs baseline: 1.0522x; 1.0522x over previous
import jax
import jax.numpy as jnp
from jax.experimental import pallas as pl
from jax.experimental.pallas import tpu as pltpu


def _gcn_kernel(length_ref, x_ref, adj_ref, w1_ref, b1_ref, w2_ref, b2_ref,
                wlin_ref, blin_ref, out_ref):
    b = pl.program_id(0)
    xb = x_ref[0]
    adjb = adj_ref[0]

    s1 = jnp.dot(xb, w1_ref[...], preferred_element_type=jnp.float32)
    h = jnp.maximum(
        jnp.dot(adjb, s1, preferred_element_type=jnp.float32) + b1_ref[0], 0.0)
    s2 = jnp.dot(h, w2_ref[...], preferred_element_type=jnp.float32)
    h2 = jnp.maximum(
        jnp.dot(adjb, s2, preferred_element_type=jnp.float32) + b2_ref[0], 0.0)

    n = adjb.shape[0]
    length = length_ref[b]
    mask = jax.lax.broadcasted_iota(jnp.int32, (n, 1), 0) < length
    pooled = (jnp.sum(jnp.where(mask, h2, 0.0), axis=0, keepdims=True)
              / length.astype(jnp.float32))
    out_ref[0] = jnp.dot(pooled, wlin_ref[...]) + blin_ref[0]


def kernel(x, adj, length, W1, b1, W2, b2, Wlin, blin):
    B, N, F = x.shape
    H1 = W1.shape[1]
    H2 = W2.shape[1]

    grid_spec = pltpu.PrefetchScalarGridSpec(
        num_scalar_prefetch=1,
        grid=(B,),
        in_specs=[
            pl.BlockSpec((1, N, F), lambda b, L: (b, 0, 0)),
            pl.BlockSpec((1, N, N), lambda b, L: (b, 0, 0)),
            pl.BlockSpec((F, H1), lambda b, L: (0, 0)),
            pl.BlockSpec((1, H1), lambda b, L: (0, 0)),
            pl.BlockSpec((H1, H2), lambda b, L: (0, 0)),
            pl.BlockSpec((1, H2), lambda b, L: (0, 0)),
            pl.BlockSpec((H2, 1), lambda b, L: (0, 0)),
            pl.BlockSpec((1, 1), lambda b, L: (0, 0)),
        ],
        out_specs=pl.BlockSpec((1, 1, 1), lambda b, L: (b, 0, 0)),
    )

    out = pl.pallas_call(
        _gcn_kernel,
        grid_spec=grid_spec,
        out_shape=jax.ShapeDtypeStruct((B, 1, 1), jnp.float32),
    )(length, x, adj, W1, b1.reshape(1, H1), W2, b2.reshape(1, H2),
      Wlin, blin.reshape(1, 1))
    return out.reshape(B, 1)


# speedup vs baseline: 1.0787x; 1.0252x over previous
import jax
import jax.numpy as jnp
from jax.experimental import pallas as pl
from jax.experimental.pallas import tpu as pltpu


def _gcn_kernel(length_ref, x_ref, adj_ref, w1_ref, b1_ref, w2_ref, b2_ref,
                wlin_ref, blin_ref, out_ref):
    b = pl.program_id(0)
    xb = x_ref[0]
    adjb = adj_ref[0]

    adj_bf = adjb.astype(jnp.bfloat16)
    s1 = jnp.dot(xb, w1_ref[...], preferred_element_type=jnp.float32)
    h = jnp.maximum(
        jnp.dot(adj_bf, s1.astype(jnp.bfloat16),
                preferred_element_type=jnp.float32) + b1_ref[0], 0.0)
    s2 = jnp.dot(h, w2_ref[...], preferred_element_type=jnp.float32)
    h2 = jnp.maximum(
        jnp.dot(adj_bf, s2.astype(jnp.bfloat16),
                preferred_element_type=jnp.float32) + b2_ref[0], 0.0)

    n = adjb.shape[0]
    length = length_ref[b]
    mask = jax.lax.broadcasted_iota(jnp.int32, (n, 1), 0) < length
    pooled = (jnp.sum(jnp.where(mask, h2, 0.0), axis=0, keepdims=True)
              / length.astype(jnp.float32))
    out_ref[0] = jnp.dot(pooled, wlin_ref[...]) + blin_ref[0]


def kernel(x, adj, length, W1, b1, W2, b2, Wlin, blin):
    B, N, F = x.shape
    H1 = W1.shape[1]
    H2 = W2.shape[1]

    grid_spec = pltpu.PrefetchScalarGridSpec(
        num_scalar_prefetch=1,
        grid=(B,),
        in_specs=[
            pl.BlockSpec((1, N, F), lambda b, L: (b, 0, 0)),
            pl.BlockSpec((1, N, N), lambda b, L: (b, 0, 0)),
            pl.BlockSpec((F, H1), lambda b, L: (0, 0)),
            pl.BlockSpec((1, H1), lambda b, L: (0, 0)),
            pl.BlockSpec((H1, H2), lambda b, L: (0, 0)),
            pl.BlockSpec((1, H2), lambda b, L: (0, 0)),
            pl.BlockSpec((H2, 1), lambda b, L: (0, 0)),
            pl.BlockSpec((1, 1), lambda b, L: (0, 0)),
        ],
        out_specs=pl.BlockSpec((1, 1, 1), lambda b, L: (b, 0, 0)),
    )

    out = pl.pallas_call(
        _gcn_kernel,
        grid_spec=grid_spec,
        out_shape=jax.ShapeDtypeStruct((B, 1, 1), jnp.float32),
    )(length, x, adj, W1, b1.reshape(1, H1), W2, b2.reshape(1, H2),
      Wlin, blin.reshape(1, 1))
    return out.reshape(B, 1)


# speedup vs baseline: 1.4741x; 1.3665x over previous
import jax
import jax.numpy as jnp
from jax.experimental import pallas as pl
from jax.experimental.pallas import tpu as pltpu


def _make_gcn_kernel(B, N, F, H1, H2):
    def body(length_ref, x_ref, adj_ref, w1_ref, b1_ref, w2_ref, b2_ref,
             wlin_ref, blin_ref, out_ref, abuf, bbuf, s2buf, sems):
        t = pl.program_id(0)
        cur = jax.lax.rem(t, 2)
        oth = jax.lax.rem(t + 1, 2)

        @pl.when(t == 0)
        def _():
            cp = pltpu.make_async_copy(adj_ref.at[0], abuf.at[0], sems.at[0])
            cp.start()
            cp.wait()

        @pl.when(t + 1 < B)
        def _():
            cp = pltpu.make_async_copy(adj_ref.at[t + 1], abuf.at[oth],
                                       sems.at[oth])
            cp.start()

        @pl.when(t > 0)
        def _():
            h2 = jnp.maximum(
                jnp.dot(bbuf[oth], s2buf[oth].astype(jnp.bfloat16),
                        preferred_element_type=jnp.float32) + b2_ref[0], 0.0)
            length = length_ref[t - 1]
            mask = jax.lax.broadcasted_iota(jnp.int32, (N, 1), 0) < length
            pooled = (jnp.sum(jnp.where(mask, h2, 0.0), axis=0, keepdims=True)
                      / length.astype(jnp.float32))
            out_ref[0] = jnp.dot(pooled, wlin_ref[...]) + blin_ref[0]

        @pl.when(jnp.logical_and(t > 0, t < B))
        def _():
            pltpu.make_async_copy(adj_ref.at[t], abuf.at[cur],
                                  sems.at[cur]).wait()

        @pl.when(t < B)
        def _():
            bbuf[cur] = abuf[cur].astype(jnp.bfloat16)
            s1 = jnp.dot(x_ref[0], w1_ref[...],
                         preferred_element_type=jnp.float32)
            h = jnp.maximum(
                jnp.dot(bbuf[cur], s1.astype(jnp.bfloat16),
                        preferred_element_type=jnp.float32) + b1_ref[0], 0.0)
            s2buf[cur] = jnp.dot(h, w2_ref[...],
                                 preferred_element_type=jnp.float32)

    return body


def kernel(x, adj, length, W1, b1, W2, b2, Wlin, blin):
    B, N, F = x.shape
    H1 = W1.shape[1]
    H2 = W2.shape[1]

    grid_spec = pltpu.PrefetchScalarGridSpec(
        num_scalar_prefetch=1,
        grid=(B + 1,),
        in_specs=[
            pl.BlockSpec((1, N, F), lambda t, L: (jnp.minimum(t, B - 1), 0, 0)),
            pl.BlockSpec(memory_space=pltpu.MemorySpace.HBM),
            pl.BlockSpec((F, H1), lambda t, L: (0, 0)),
            pl.BlockSpec((1, H1), lambda t, L: (0, 0)),
            pl.BlockSpec((H1, H2), lambda t, L: (0, 0)),
            pl.BlockSpec((1, H2), lambda t, L: (0, 0)),
            pl.BlockSpec((H2, 1), lambda t, L: (0, 0)),
            pl.BlockSpec((1, 1), lambda t, L: (0, 0)),
        ],
        out_specs=pl.BlockSpec((1, 1, 1),
                               lambda t, L: (jnp.maximum(t - 1, 0), 0, 0)),
        scratch_shapes=[
            pltpu.VMEM((2, N, N), jnp.float32),
            pltpu.VMEM((2, N, N), jnp.bfloat16),
            pltpu.VMEM((2, N, H2), jnp.float32),
            pltpu.SemaphoreType.DMA((2,)),
        ],
    )

    out = pl.pallas_call(
        _make_gcn_kernel(B, N, F, H1, H2),
        grid_spec=grid_spec,
        out_shape=jax.ShapeDtypeStruct((B, 1, 1), jnp.float32),
    )(length, x, adj, W1, b1.reshape(1, H1), W2, b2.reshape(1, H2),
      Wlin, blin.reshape(1, 1))
    return out.reshape(B, 1)
